# Initial kernel scaffold; baseline (speedup 1.0000x reference)
import functools

import jax
import jax.numpy as jnp
from jax import lax
from jax.experimental import pallas as pl
from jax.experimental.pallas import tpu as pltpu
from jax.experimental.pallas import tpu_sc as plsc

E = 320000
D = 128
S = 10000

NC = 2
NS = 16
NW = NC * NS
EPT = E // NW
BLK = 128
NFULL = EPT // BLK
REM = EPT - NFULL * BLK
ROWS_PER_TILE = S // NS
LANES = 16

_mesh = plsc.VectorSubcoreMesh(core_axis_name="c", subcore_axis_name="s")


@functools.partial(
    pl.kernel,
    mesh=_mesh,
    out_type=jax.ShapeDtypeStruct((NC, S, D), jnp.float32),
    scratch_types=[
        pltpu.VMEM((BLK,), jnp.int32),
        pltpu.VMEM((BLK, D), jnp.float32),
        pltpu.VMEM((REM,), jnp.int32),
        pltpu.VMEM((REM, D), jnp.float32),
        pltpu.VMEM_SHARED((S, D), jnp.float32),
    ],
)
def _sc_segment_sum(x_hbm, idx_hbm, out_hbm, idx_v, x_v, idx_r, x_r, accum):
    c = lax.axis_index("c")
    s = lax.axis_index("s")
    wid = c * NS + s

    zero16 = jnp.zeros((LANES,), jnp.float32)

    def zrow(r, carry):
        for k in range(D // LANES):
            x_v[r, pl.ds(k * LANES, LANES)] = zero16
        return carry

    lax.fori_loop(0, BLK, zrow, 0)

    row0 = s * ROWS_PER_TILE
    for i in range(5):
        pltpu.sync_copy(
            x_v.at[pl.ds(0, 125)],
            accum.at[pl.ds(row0 + i * 125, 125)],
        )
    plsc.subcore_barrier()

    ebase = wid * EPT

    def body(j, carry):
        off = ebase + j * BLK
        pltpu.sync_copy(idx_hbm.at[pl.ds(off, BLK)], idx_v)
        pltpu.sync_copy(x_hbm.at[pl.ds(off, BLK)], x_v)
        pltpu.sync_copy(x_v, accum.at[idx_v], add=True)
        return carry

    lax.fori_loop(0, NFULL, body, 0)

    off = ebase + NFULL * BLK
    pltpu.sync_copy(idx_hbm.at[pl.ds(off, REM)], idx_r)
    pltpu.sync_copy(x_hbm.at[pl.ds(off, REM)], x_r)
    pltpu.sync_copy(x_r, accum.at[idx_r], add=True)

    plsc.subcore_barrier()

    pltpu.sync_copy(
        accum.at[pl.ds(row0, ROWS_PER_TILE)],
        out_hbm.at[c].at[pl.ds(row0, ROWS_PER_TILE)],
    )


_RB = 1000


def _combine_body(p_ref, o_ref):
    o_ref[...] = p_ref[0] + p_ref[1]


def _combine(partials):
    return pl.pallas_call(
        _combine_body,
        grid=(S // _RB,),
        in_specs=[pl.BlockSpec((NC, _RB, D), lambda i: (0, i, 0))],
        out_specs=pl.BlockSpec((_RB, D), lambda i: (i, 0)),
        out_shape=jax.ShapeDtypeStruct((S, D), jnp.float32),
    )(partials)


def kernel(x, index):
    idx32 = index.astype(jnp.int32)
    partials = _sc_segment_sum(x, idx32)
    return _combine(partials)


# speedup vs baseline: 4.5576x; 4.5576x over previous
import functools

import jax
import jax.numpy as jnp
from jax import lax
from jax.experimental import pallas as pl
from jax.experimental.pallas import tpu as pltpu
from jax.experimental.pallas import tpu_sc as plsc

E = 320000
D = 128
S = 10000
SPAD = 10240

NC = 2
NS = 16
NW = NC * NS
EPT = E // NW
BLK = 128
NFULL = EPT // BLK
REM = EPT - NFULL * BLK
ROWS_PER_TILE = SPAD // NS
LANES = 16

_mesh = plsc.VectorSubcoreMesh(core_axis_name="c", subcore_axis_name="s")


@functools.partial(
    pl.kernel,
    mesh=_mesh,
    out_type=jax.ShapeDtypeStruct((NC, SPAD, D), jnp.float32),
    scratch_types=[
        pltpu.VMEM((BLK,), jnp.int32),
        pltpu.VMEM((BLK, D), jnp.float32),
        pltpu.VMEM((REM,), jnp.int32),
        pltpu.VMEM((REM, D), jnp.float32),
        pltpu.VMEM_SHARED((SPAD, D), jnp.float32),
    ],
)
def _sc_segment_sum(x_hbm, idx_hbm, out_hbm, idx_v, x_v, idx_r, x_r, accum):
    c = lax.axis_index("c")
    s = lax.axis_index("s")
    wid = c * NS + s

    zero16 = jnp.zeros((LANES,), jnp.float32)

    def zrow(r, carry):
        for k in range(D // LANES):
            x_v[r, pl.ds(k * LANES, LANES)] = zero16
        return carry

    lax.fori_loop(0, BLK, zrow, 0)

    row0 = pl.multiple_of(s * ROWS_PER_TILE, 8)
    for i in range(ROWS_PER_TILE // BLK):
        pltpu.sync_copy(
            x_v,
            accum.at[pl.ds(pl.multiple_of(row0 + i * BLK, 8), BLK)],
        )
    plsc.subcore_barrier()

    ebase = wid * EPT

    def body(j, carry):
        off = pl.multiple_of(ebase + j * BLK, 8)
        pltpu.sync_copy(idx_hbm.at[pl.ds(off, BLK)], idx_v)
        pltpu.sync_copy(x_hbm.at[pl.ds(off, BLK)], x_v)
        pltpu.sync_copy(x_v, accum.at[idx_v], add=True)
        return carry

    lax.fori_loop(0, NFULL, body, 0)

    off = pl.multiple_of(ebase + NFULL * BLK, 8)
    pltpu.sync_copy(idx_hbm.at[pl.ds(off, REM)], idx_r)
    pltpu.sync_copy(x_hbm.at[pl.ds(off, REM)], x_r)
    pltpu.sync_copy(x_r, accum.at[idx_r], add=True)

    plsc.subcore_barrier()

    pltpu.sync_copy(
        accum.at[pl.ds(row0, ROWS_PER_TILE)],
        out_hbm.at[c].at[pl.ds(row0, ROWS_PER_TILE)],
    )


_RB = 1000


def _combine_body(p_ref, o_ref):
    o_ref[...] = p_ref[0] + p_ref[1]


def _combine(partials):
    return pl.pallas_call(
        _combine_body,
        grid=(S // _RB,),
        in_specs=[pl.BlockSpec((NC, _RB, D), lambda i: (0, i, 0))],
        out_specs=pl.BlockSpec((_RB, D), lambda i: (i, 0)),
        out_shape=jax.ShapeDtypeStruct((S, D), jnp.float32),
    )(partials)


def kernel(x, index):
    idx32 = index.astype(jnp.int32)
    partials = _sc_segment_sum(x, idx32)
    return _combine(partials)


# speedup vs baseline: 8.2503x; 1.8102x over previous
import functools

import jax
import jax.numpy as jnp
from jax import lax
from jax.experimental import pallas as pl
from jax.experimental.pallas import tpu as pltpu
from jax.experimental.pallas import tpu_sc as plsc

E = 320000
D = 128
S = 10000
SPAD = 10112

NC = 2
NS = 16
NW = NC * NS

BLK = 128
NBUF = 3
NB_LO = 78
NB_HI = 79

ROWS_PER_TILE = SPAD // NS
LANES = 16

_mesh = plsc.VectorSubcoreMesh(core_axis_name="c", subcore_axis_name="s")


@functools.partial(
    pl.kernel,
    mesh=_mesh,
    out_type=jax.ShapeDtypeStruct((NC, SPAD, D), jnp.float32),
    scratch_types=[
        pltpu.VMEM((NBUF, BLK, D), jnp.float32),
        pltpu.VMEM((BLK,), jnp.int32),
        pltpu.VMEM((BLK,), jnp.int32),
        pltpu.VMEM((BLK,), jnp.int32),
        pltpu.SemaphoreType.DMA,
        pltpu.SemaphoreType.DMA,
        pltpu.SemaphoreType.DMA,
        pltpu.VMEM_SHARED((SPAD, D), jnp.float32),
    ],
)
def _sc_segment_sum(x_hbm, idx_hbm, out_hbm, x_v,
                    i0, i1, i2, sem0, sem1, sem2, accum):
    c = lax.axis_index("c")
    s = lax.axis_index("s")
    wid = c * NS + s

    ib = [i0, i1, i2]
    sems = [sem0, sem1, sem2]

    base = jnp.where(wid < 4, wid * (NB_HI * BLK),
                     4 * (NB_HI * BLK) + (wid - 4) * (NB_LO * BLK))
    nb = jnp.where(wid < 4, NB_HI, NB_LO)

    zero16 = jnp.zeros((LANES,), jnp.float32)

    def zrow(r, carry):
        for k in range(D // LANES):
            x_v[0, r, pl.ds(k * LANES, LANES)] = zero16
        return carry

    lax.fori_loop(0, BLK, zrow, 0)

    row0 = pl.multiple_of(s * ROWS_PER_TILE, 8)
    for i in range(4):
        pltpu.sync_copy(
            x_v.at[0],
            accum.at[pl.ds(pl.multiple_of(row0 + i * BLK, 8), BLK)],
        )
    pltpu.sync_copy(
        x_v.at[0, pl.ds(0, ROWS_PER_TILE - 4 * BLK)],
        accum.at[pl.ds(pl.multiple_of(row0 + 4 * BLK, 8), ROWS_PER_TILE - 4 * BLK)],
    )

    def load_descs(g, b):
        off = pl.multiple_of(base + g * BLK, 8)
        return [
            pltpu.make_async_copy(idx_hbm.at[pl.ds(off, BLK)], ib[b], sems[b]),
            pltpu.make_async_copy(x_hbm.at[pl.ds(off, BLK)], x_v.at[b], sems[b]),
        ]

    def start_load(g, b):
        for d in load_descs(g, b):
            d.start()

    def wait_load(g, b):
        for d in load_descs(g, b):
            d.wait()

    def scatter(b):
        pltpu.sync_copy(x_v.at[b], accum.at[ib[b]], add=True)

    start_load(0, 0)
    start_load(1, 1)
    plsc.subcore_barrier()

    def body(jo, carry):
        for b in range(NBUF):
            g = jo * NBUF + b

            @pl.when(g + 2 < nb)
            def _():
                start_load(g + 2, (b + 2) % NBUF)

            wait_load(g, b)
            scatter(b)
        return carry

    lax.fori_loop(0, NB_LO // NBUF, body, 0)

    @pl.when(nb == NB_HI)
    def _():
        wait_load(NB_LO, NB_LO % NBUF)
        scatter(NB_LO % NBUF)

    plsc.subcore_barrier()

    pltpu.sync_copy(
        accum.at[pl.ds(row0, ROWS_PER_TILE)],
        out_hbm.at[c].at[pl.ds(row0, ROWS_PER_TILE)],
    )


_RB = 1000


def _combine_body(p_ref, o_ref):
    o_ref[...] = p_ref[0] + p_ref[1]


def _combine(partials):
    return pl.pallas_call(
        _combine_body,
        grid=(S // _RB,),
        in_specs=[pl.BlockSpec((NC, _RB, D), lambda i: (0, i, 0))],
        out_specs=pl.BlockSpec((_RB, D), lambda i: (i, 0)),
        out_shape=jax.ShapeDtypeStruct((S, D), jnp.float32),
    )(partials)


def kernel(x, index):
    idx32 = index.astype(jnp.int32)
    partials = _sc_segment_sum(x, idx32)
    return _combine(partials)
